# baseline (device time: 70175 ns/iter reference)
import jax
import jax.numpy as jnp
from jax import lax
from jax.experimental import pallas as pl
from jax.experimental.pallas import tpu as pltpu

N_DEV = 4
M = 2048
D = 512
H = 1024
E = 32
E_LOC = E // N_DEV
BLK = M // N_DEV


def kernel(x, router_W, route_idx, expert_W, shared_W):
    def body(x_ref, rw_ref, idx_ref, ew_ref, sw_ref, out_ref,
             w_ref, acc_ref, recv_ref, xb16_ref, ewb_ref, swb_ref,
             send_sems, recv_sems):
        my = lax.axis_index("i")
        left = (my + N_DEV - 1) % N_DEV
        right = (my + 1) % N_DEV

        barrier_sem = pltpu.get_barrier_semaphore()
        for nbr in (left, right):
            pl.semaphore_signal(barrier_sem, inc=1, device_id=(nbr,),
                                device_id_type=pl.DeviceIdType.MESH)
        pl.semaphore_wait(barrier_sem, 2)

        scores = jnp.dot(x_ref[:, :], rw_ref[:, :],
                         preferred_element_type=jnp.float32)
        smax = jnp.max(scores, axis=1, keepdims=True)
        pr = jnp.exp(scores - smax)
        pr = pr / jnp.sum(pr, axis=1, keepdims=True)
        idx = idx_ref[:, :]
        eids = lax.broadcasted_iota(jnp.int32, (M, E), 1)
        sel = jnp.sum(jnp.where(eids == idx, pr, 0.0), axis=1,
                      keepdims=True)
        loc = my * E_LOC + lax.broadcasted_iota(jnp.int32, (1, E_LOC), 1)
        w_ref[:, :] = jnp.where(idx == loc, sel, 0.0).astype(jnp.bfloat16)

        xb16_ref[:, :] = x_ref[:, :].astype(jnp.bfloat16)
        ewb_ref[:, :, :] = ew_ref[:, :, :].astype(jnp.bfloat16)
        swb_ref[:, :] = sw_ref[:, :].astype(jnp.bfloat16)

        def partial_block(b):
            xb = xb16_ref[pl.ds(b * BLK, BLK), :]
            wb = w_ref[pl.ds(b * BLK, BLK), :]
            acc = jnp.zeros((BLK, H), jnp.float32)
            for le in range(E_LOC):
                acc = acc + jnp.dot(xb * wb[:, le:le + 1], ewb_ref[le],
                                    preferred_element_type=jnp.float32)
            return acc

        rdmas = []
        acc_ref[0] = partial_block((my + N_DEV - 1) % N_DEV).astype(
            jnp.bfloat16)
        for h in range(N_DEV - 1):
            rdma = pltpu.make_async_remote_copy(
                src_ref=acc_ref.at[h],
                dst_ref=recv_ref.at[h],
                send_sem=send_sems.at[h],
                recv_sem=recv_sems.at[h],
                device_id=(right,),
                device_id_type=pl.DeviceIdType.MESH,
            )
            rdma.start()
            rdmas.append(rdma)
            if h < N_DEV - 2:
                nxt = partial_block((my + N_DEV - 2 - h) % N_DEV)
            else:
                nxt = partial_block(my)
                xm = xb16_ref[pl.ds(my * BLK, BLK), :]
                nxt = nxt + jnp.dot(xm, swb_ref[:, :],
                                    preferred_element_type=jnp.float32)
            rdma.wait_recv()
            if h < N_DEV - 2:
                acc_ref[h + 1] = (
                    nxt + recv_ref[h].astype(jnp.float32)
                ).astype(jnp.bfloat16)
            else:
                out_ref[:, :] = nxt + recv_ref[h].astype(jnp.float32)
        for rdma in rdmas:
            rdma.wait_send()

    return pl.pallas_call(
        body,
        out_shape=jax.ShapeDtypeStruct((BLK, H), jnp.float32),
        in_specs=[pl.BlockSpec(memory_space=pltpu.VMEM)] * 5,
        out_specs=pl.BlockSpec(memory_space=pltpu.VMEM),
        scratch_shapes=[
            pltpu.VMEM((M, E_LOC), jnp.bfloat16),
            pltpu.VMEM((N_DEV - 1, BLK, H), jnp.bfloat16),
            pltpu.VMEM((N_DEV - 1, BLK, H), jnp.bfloat16),
            pltpu.VMEM((M, D), jnp.bfloat16),
            pltpu.VMEM((E_LOC, D, H), jnp.bfloat16),
            pltpu.VMEM((D, H), jnp.bfloat16),
            pltpu.SemaphoreType.DMA((N_DEV - 1,)),
            pltpu.SemaphoreType.DMA((N_DEV - 1,)),
        ],
        compiler_params=pltpu.CompilerParams(
            collective_id=0,
            vmem_limit_bytes=56 * 1024 * 1024,
        ),
    )(x, router_W, route_idx, expert_W, shared_W)


# device time: 57517 ns/iter; 1.2201x vs baseline; 1.2201x over previous
import jax
import jax.numpy as jnp
from jax import lax
from jax.experimental import pallas as pl
from jax.experimental.pallas import tpu as pltpu

N_DEV = 4
M = 2048
D = 512
H = 1024
E = 32
E_LOC = E // N_DEV
BLK = M // N_DEV
NCH = 2
CH = BLK // NCH


def kernel(x, router_W, route_idx, expert_W, shared_W):
    def body(x_ref, rw_ref, idx_ref, ew_ref, sw_ref, out_ref,
             w_ref, acc_ref, recv_ref, send_sems, recv_sems):
        my = lax.axis_index("i")
        left = (my + N_DEV - 1) % N_DEV
        right = (my + 1) % N_DEV

        barrier_sem = pltpu.get_barrier_semaphore()
        for nbr in (left, right):
            pl.semaphore_signal(barrier_sem, inc=1, device_id=(nbr,),
                                device_id_type=pl.DeviceIdType.MESH)
        pl.semaphore_wait(barrier_sem, 2)

        scores = jnp.dot(x_ref[:, :], rw_ref[:, :],
                         preferred_element_type=jnp.float32)
        smax = jnp.max(scores, axis=1, keepdims=True)
        pr = jnp.exp(scores - smax)
        pr = pr / jnp.sum(pr, axis=1, keepdims=True)
        idx = idx_ref[:, :]
        eids = lax.broadcasted_iota(jnp.int32, (M, E), 1)
        sel = jnp.sum(jnp.where(eids == idx, pr, 0.0), axis=1,
                      keepdims=True)
        loc = my * E_LOC + lax.broadcasted_iota(jnp.int32, (1, E_LOC), 1)
        w_ref[:, :] = jnp.where(idx == loc, sel, 0.0)

        def partial(b, c):
            off = b * BLK + c * CH
            xc = x_ref[pl.ds(off, CH), :]
            wc = w_ref[pl.ds(off, CH), :]
            acc = jnp.zeros((CH, H), jnp.float32)
            for le in range(E_LOC):
                acc = acc + jnp.dot(xc * wc[:, le:le + 1], ew_ref[le],
                                    preferred_element_type=jnp.float32)
            return acc

        def start_hop(h, c):
            rdma = pltpu.make_async_remote_copy(
                src_ref=acc_ref.at[h, c],
                dst_ref=recv_ref.at[h, c],
                send_sem=send_sems.at[h, c],
                recv_sem=recv_sems.at[h, c],
                device_id=(right,),
                device_id_type=pl.DeviceIdType.MESH,
            )
            rdma.start()
            return rdma

        rdmas = {}
        for c in range(NCH):
            acc_ref[0, c] = partial((my + N_DEV - 1) % N_DEV, c).astype(
                jnp.bfloat16)
            rdmas[0, c] = start_hop(0, c)
        for h in range(N_DEV - 1):
            b_next = (my + N_DEV - 2 - h) % N_DEV
            for c in range(NCH):
                if h < N_DEV - 2:
                    nxt = partial(b_next, c)
                else:
                    nxt = partial(my, c)
                    xm = x_ref[pl.ds(my * BLK + c * CH, CH), :]
                    nxt = nxt + jnp.dot(xm, sw_ref[:, :],
                                        preferred_element_type=jnp.float32)
                rdmas[h, c].wait_recv()
                if h < N_DEV - 2:
                    acc_ref[h + 1, c] = (
                        nxt + recv_ref[h, c].astype(jnp.float32)
                    ).astype(jnp.bfloat16)
                    rdmas[h + 1, c] = start_hop(h + 1, c)
                else:
                    out_ref[pl.ds(c * CH, CH), :] = (
                        nxt + recv_ref[h, c].astype(jnp.float32))
        for rdma in rdmas.values():
            rdma.wait_send()

    return pl.pallas_call(
        body,
        out_shape=jax.ShapeDtypeStruct((BLK, H), jnp.float32),
        in_specs=[pl.BlockSpec(memory_space=pltpu.VMEM)] * 5,
        out_specs=pl.BlockSpec(memory_space=pltpu.VMEM),
        scratch_shapes=[
            pltpu.VMEM((M, E_LOC), jnp.float32),
            pltpu.VMEM((N_DEV - 1, NCH, CH, H), jnp.bfloat16),
            pltpu.VMEM((N_DEV - 1, NCH, CH, H), jnp.bfloat16),
            pltpu.SemaphoreType.DMA((N_DEV - 1, NCH)),
            pltpu.SemaphoreType.DMA((N_DEV - 1, NCH)),
        ],
        compiler_params=pltpu.CompilerParams(
            collective_id=0,
            vmem_limit_bytes=48 * 1024 * 1024,
        ),
    )(x, router_W, route_idx, expert_W, shared_W)
